# baseline (speedup 1.0000x reference)
import functools

import jax
import jax.numpy as jnp
from jax import lax
from jax.experimental import pallas as pl
from jax.experimental.pallas import tpu as pltpu
from jax.experimental.pallas import tpu_sc as plsc

_NC = 2
_NS = 16
_NW = _NC * _NS

_B = 16384
_L = 39
_D = 32
_RPW = _B // _NW


@functools.partial(
    pl.kernel,
    out_type=jax.ShapeDtypeStruct((_L, 4, 128, 8, 128), jnp.float32),
    mesh=plsc.VectorSubcoreMesh(
        core_axis_name="c", subcore_axis_name="s",
        num_cores=_NC, num_subcores=_NS,
    ),
    scratch_types=[
        pltpu.VMEM((_RPW, _L), jnp.int32),
        pltpu.VMEM((_L, _RPW), jnp.int32),
        pltpu.VMEM((_RPW, _D), jnp.float32),
        pltpu.VMEM((4, 4, 8, 128), jnp.float32),
        pltpu.SemaphoreType.DMA,
    ],
    compiler_params=pltpu.CompilerParams(
        use_tc_tiling_on_sc=False, needs_layout_passes=False),
)
def _emb_gather(idx_hbm, table_hbm, out_hbm, idx_v, idx_t, rows_v, obuf, sem):
    wid = lax.axis_index("s") * _NC + lax.axis_index("c")
    base = wid * _RPW
    tb0 = wid * 4
    lanes = lax.iota(jnp.int32, 16)

    pltpu.sync_copy(idx_hbm.at[pl.ds(base, _RPW)], idx_v)

    def idx_step(i, carry):
        l = i // (_RPW // 16)
        g = i % (_RPW // 16)
        v = plsc.load_gather(idx_v, [g * 16 + lanes, jnp.full((16,), l, jnp.int32)])
        idx_t[l, pl.ds(g * 16, 16)] = v
        return carry

    lax.fori_loop(0, _L * (_RPW // 16), idx_step, 0)

    def l_step(l, carry):
        pltpu.async_copy(table_hbm.at[idx_t.at[l]], rows_v, sem).wait()

        def t_step(i, c2):
            r = i // 8
            g = i % 8
            td = r // 32
            tb = (r // 8) % 4
            d8 = r % 8
            v = plsc.load_gather(
                rows_v,
                [tb * 128 + g * 16 + lanes,
                 jnp.full((16,), td * 8 + d8, jnp.int32)])
            obuf[td, tb, d8, pl.ds(g * 16, 16)] = v
            return c2

        lax.fori_loop(0, 1024, t_step, 0)

        for td in range(4):
            pltpu.sync_copy(obuf.at[td], out_hbm.at[l, td, pl.ds(tb0, 4)])
        return carry

    lax.fori_loop(0, _L, l_step, 0)


def kernel(DNA_x, tf_x, W):
    o5 = _emb_gather(tf_x.astype(jnp.int32), W)
    emb = o5.transpose((2, 4, 0, 1, 3)).reshape(_B, _L, _D)
    return (DNA_x, emb)


# speedup vs baseline: 1.1459x; 1.1459x over previous
import functools

import jax
import jax.numpy as jnp
from jax import lax
from jax.experimental import pallas as pl
from jax.experimental import layout as jlayout
from jax.experimental.pallas import tpu as pltpu
from jax.experimental.pallas import tpu_sc as plsc

_NC = 2
_NS = 16
_NW = _NC * _NS

_B = 16384
_L = 39
_D = 32
_RPW = _B // _NW


@functools.partial(
    pl.kernel,
    out_type=jax.ShapeDtypeStruct((_L, 4, 128, 8, 128), jnp.float32),
    mesh=plsc.VectorSubcoreMesh(
        core_axis_name="c", subcore_axis_name="s",
        num_cores=_NC, num_subcores=_NS,
    ),
    scratch_types=[
        pltpu.VMEM((_RPW, _L), jnp.int32),
        pltpu.VMEM((_L, _RPW), jnp.int32),
        pltpu.VMEM((_RPW, _D), jnp.float32),
        pltpu.VMEM((_RPW, _D), jnp.float32),
        pltpu.VMEM((4, 4, 8, 128), jnp.float32),
        pltpu.VMEM((4, 4, 8, 128), jnp.float32),
        pltpu.SemaphoreType.DMA,
        pltpu.SemaphoreType.DMA,
        pltpu.SemaphoreType.DMA,
        pltpu.SemaphoreType.DMA,
    ],
    compiler_params=pltpu.CompilerParams(
        use_tc_tiling_on_sc=False, needs_layout_passes=False),
)
def _emb_gather(idx_hbm, table_hbm, out_hbm, idx_v, idx_t, rows0, rows1,
                obuf0, obuf1, sem_g0, sem_g1, sem_s0, sem_s1):
    wid = lax.axis_index("s") * _NC + lax.axis_index("c")
    base = wid * _RPW
    tb0 = wid * 4
    lanes = lax.iota(jnp.int32, 16)

    pltpu.sync_copy(idx_hbm.at[pl.ds(base, _RPW)], idx_v)

    def idx_step(l, carry):
        for g in range(_RPW // 16):
            v = plsc.load_gather(
                idx_v, [g * 16 + lanes, jnp.full((16,), l, jnp.int32)])
            idx_t[l, pl.ds(g * 16, 16)] = v
        return carry

    lax.fori_loop(0, _L, idx_step, 0)

    def gather_fire(l, buf, sem):
        pltpu.async_copy(table_hbm.at[idx_t.at[l]], buf, sem)

    def gather_wait(l, buf, sem):
        pltpu.make_async_copy(table_hbm.at[idx_t.at[l]], buf, sem).wait()

    def scatter_fire(l, obuf, sem):
        for td in range(4):
            pltpu.async_copy(obuf.at[td], out_hbm.at[l, td, pl.ds(tb0, 4)], sem)

    def scatter_wait(l, obuf, sem):
        for td in range(4):
            pltpu.make_async_copy(
                obuf.at[td], out_hbm.at[l, td, pl.ds(tb0, 4)], sem).wait()

    def transpose(rows, obuf):
        def t_step(q, c2):
            for rr in range(4):
                r = q * 4 + rr
                td = r // 32
                tb = (r // 8) % 4
                d8 = r % 8
                col = jnp.full((16,), td * 8 + d8, jnp.int32)
                for g in range(8):
                    v = plsc.load_gather(
                        rows, [tb * 128 + g * 16 + lanes, col])
                    obuf[td, tb, d8, pl.ds(g * 16, 16)] = v
            return c2

        lax.fori_loop(0, 32, t_step, 0)

    gather_fire(0, rows0, sem_g0)

    def step(k, carry):
        l0 = 2 * k
        l1 = l0 + 1

        gather_fire(l1, rows1, sem_g1)
        gather_wait(l0, rows0, sem_g0)

        @pl.when(k > 0)
        def _():
            scatter_wait(l0 - 2, obuf0, sem_s0)

        transpose(rows0, obuf0)
        scatter_fire(l0, obuf0, sem_s0)

        gather_fire(l0 + 2, rows0, sem_g0)

        gather_wait(l1, rows1, sem_g1)

        @pl.when(k > 0)
        def _():
            scatter_wait(l1 - 2, obuf1, sem_s1)

        transpose(rows1, obuf1)
        scatter_fire(l1, obuf1, sem_s1)
        return carry

    lax.fori_loop(0, _L // 2, step, 0)
    scatter_wait(_L - 3, obuf0, sem_s0)
    gather_wait(_L - 1, rows0, sem_g0)
    transpose(rows0, obuf0)
    scatter_fire(_L - 1, obuf0, sem_s0)
    scatter_wait(_L - 2, obuf1, sem_s1)
    scatter_wait(_L - 1, obuf0, sem_s0)


def kernel(DNA_x, tf_x, W):
    Wt = jlayout.with_layout_constraint(
        W.reshape(250000, 128),
        jlayout.Layout(major_to_minor=(0, 1), tiling=((8, 128),)))
    o5 = _emb_gather(tf_x.astype(jnp.int32), Wt.reshape(1000000, 32))
    emb = o5.transpose((2, 4, 0, 1, 3)).reshape(_B, _L, _D)
    return (DNA_x, emb)


# speedup vs baseline: 1.6844x; 1.4699x over previous
import functools

import jax
import jax.numpy as jnp
from jax import lax
from jax.experimental import pallas as pl
from jax.experimental import layout as jlayout
from jax.experimental.pallas import tpu as pltpu
from jax.experimental.pallas import tpu_sc as plsc

_NC = 2
_NS = 16
_NW = _NC * _NS

_B = 16384
_L = 39
_D = 32
_RPW = _B // _NW


@functools.partial(
    pl.kernel,
    out_type=jax.ShapeDtypeStruct((_L, 4, 128, 8, 128), jnp.float32),
    mesh=plsc.VectorSubcoreMesh(
        core_axis_name="c", subcore_axis_name="s",
        num_cores=_NC, num_subcores=_NS,
    ),
    scratch_types=[
        pltpu.VMEM((_RPW, _L), jnp.int32),
        pltpu.VMEM((_L, _RPW), jnp.int32),
        pltpu.VMEM((_RPW, _D), jnp.float32),
        pltpu.VMEM((_RPW, _D), jnp.float32),
        pltpu.VMEM((4, 5, 8, 129), jnp.float32),
        pltpu.VMEM((4, 5, 8, 129), jnp.float32),
        pltpu.SemaphoreType.DMA,
        pltpu.SemaphoreType.DMA,
        pltpu.SemaphoreType.DMA,
        pltpu.SemaphoreType.DMA,
    ],
    compiler_params=pltpu.CompilerParams(
        use_tc_tiling_on_sc=False, needs_layout_passes=False),
)
def _emb_gather(idx_hbm, table_hbm, out_hbm, idx_v, idx_t, rows0, rows1,
                obuf0, obuf1, sem_g0, sem_g1, sem_s0, sem_s1):
    wid = lax.axis_index("s") * _NC + lax.axis_index("c")
    base = wid * _RPW
    tb0 = wid * 4
    lanes = lax.iota(jnp.int32, 16)

    pltpu.sync_copy(idx_hbm.at[pl.ds(base, _RPW)], idx_v)

    def idx_step(l, carry):
        for g in range(_RPW // 16):
            v = plsc.load_gather(
                idx_v, [g * 16 + lanes, jnp.full((16,), l, jnp.int32)])
            idx_t[l, pl.ds(g * 16, 16)] = v
        return carry

    lax.fori_loop(0, _L, idx_step, 0)

    def gather_fire(l, buf, sem):
        pltpu.async_copy(table_hbm.at[idx_t.at[l]], buf, sem)

    def gather_wait(l, buf, sem):
        pltpu.make_async_copy(table_hbm.at[idx_t.at[l]], buf, sem).wait()

    def scatter_fire(l, obuf, sem):
        for td in range(4):
            for tb in range(4):
                pltpu.async_copy(
                    obuf.at[td, tb, pl.ds(0, 8), pl.ds(0, 128)],
                    out_hbm.at[l, td, tb0 + tb], sem)

    def scatter_wait(l, obuf, sem):
        for td in range(4):
            for tb in range(4):
                pltpu.make_async_copy(
                    obuf.at[td, tb, pl.ds(0, 8), pl.ds(0, 128)],
                    out_hbm.at[l, td, tb0 + tb], sem).wait()

    td_lo = lanes >> 3
    td_hi = td_lo + 2
    d8_pat = lanes & 7
    ones16 = jnp.full((16,), 1, jnp.int32)

    def transpose(rows, obuf):
        def t_step(q, c2):
            for bb in range(4):
                b = q * 4 + bb
                tb_v = ones16 * (b >> 7)
                b128_v = ones16 * (b & 127)
                v0 = rows[b, pl.ds(0, 16)]
                plsc.store_scatter(obuf, [td_lo, tb_v, d8_pat, b128_v], v0)
                v1 = rows[b, pl.ds(16, 16)]
                plsc.store_scatter(obuf, [td_hi, tb_v, d8_pat, b128_v], v1)
            return c2

        lax.fori_loop(0, _RPW // 4, t_step, 0)

    gather_fire(0, rows0, sem_g0)

    def step(k, carry):
        l0 = 2 * k
        l1 = l0 + 1

        gather_fire(l1, rows1, sem_g1)
        gather_wait(l0, rows0, sem_g0)

        @pl.when(k > 0)
        def _():
            scatter_wait(l0 - 2, obuf0, sem_s0)

        transpose(rows0, obuf0)
        scatter_fire(l0, obuf0, sem_s0)

        gather_fire(l0 + 2, rows0, sem_g0)

        gather_wait(l1, rows1, sem_g1)

        @pl.when(k > 0)
        def _():
            scatter_wait(l1 - 2, obuf1, sem_s1)

        transpose(rows1, obuf1)
        scatter_fire(l1, obuf1, sem_s1)
        return carry

    lax.fori_loop(0, _L // 2, step, 0)
    scatter_wait(_L - 3, obuf0, sem_s0)
    gather_wait(_L - 1, rows0, sem_g0)
    transpose(rows0, obuf0)
    scatter_fire(_L - 1, obuf0, sem_s0)
    scatter_wait(_L - 2, obuf1, sem_s1)
    scatter_wait(_L - 1, obuf0, sem_s0)


def kernel(DNA_x, tf_x, W):
    Wt = jlayout.with_layout_constraint(
        W.reshape(250000, 128),
        jlayout.Layout(major_to_minor=(0, 1), tiling=((8, 128),)))
    o5 = _emb_gather(tf_x.astype(jnp.int32), Wt.reshape(1000000, 32))
    emb = o5.transpose((2, 4, 0, 1, 3)).reshape(_B, _L, _D)
    return (DNA_x, emb)
